# baseline (device time: 182545 ns/iter reference)
import jax
import jax.numpy as jnp
from jax import lax
from jax.experimental import pallas as pl
from jax.experimental.pallas import tpu as pltpu


def kernel(x):
    xs = x[0, 0].astype(jnp.bfloat16)
    m, n = xs.shape
    q = m // 4
    h = q // 2

    def body(
        x_ref,
        out_ref,
        recv_a1,
        recv_b1,
        recv_a2,
        recv_b2,
        acc_a,
        acc_b,
        send_sems,
        recv_sems,
    ):
        mx = lax.axis_index("x")
        my = lax.axis_index("y")
        x_nbr = (1 - mx, my)
        y_nbr = (mx, 1 - my)

        a_mine = mx * q
        a_theirs = (1 - mx) * q
        b_mine = 2 * q + my * q
        b_theirs = 2 * q + (1 - my) * q

        barrier = pltpu.get_barrier_semaphore()
        for nbr in (x_nbr, y_nbr):
            pl.semaphore_signal(
                barrier, inc=1, device_id=nbr,
                device_id_type=pl.DeviceIdType.MESH,
            )
        pl.semaphore_wait(barrier, 2)

        def exch(src, dst, sem, nbr):
            return pltpu.make_async_remote_copy(
                src_ref=src, dst_ref=dst,
                send_sem=send_sems.at[sem], recv_sem=recv_sems.at[sem],
                device_id=nbr, device_id_type=pl.DeviceIdType.MESH,
            )

        p1a = [
            exch(x_ref.at[pl.ds(a_theirs + c * h, h), :],
                 recv_a1.at[pl.ds(c * h, h), :], 0 + c, x_nbr)
            for c in range(2)
        ]
        p1b = [
            exch(x_ref.at[pl.ds(b_theirs + c * h, h), :],
                 recv_b1.at[pl.ds(c * h, h), :], 2 + c, y_nbr)
            for c in range(2)
        ]
        for r in (*p1a, *p1b):
            r.start()

        p2a = [
            exch(acc_a.at[pl.ds(c * h, h), :],
                 recv_a2.at[pl.ds(c * h, h), :], 4 + c, y_nbr)
            for c in range(2)
        ]
        p2b = [
            exch(acc_b.at[pl.ds(c * h, h), :],
                 recv_b2.at[pl.ds(c * h, h), :], 6 + c, x_nbr)
            for c in range(2)
        ]
        for c in range(2):
            s = pl.ds(c * h, h)
            p1a[c].wait()
            acc_a[s, :] = x_ref[pl.ds(a_mine + c * h, h), :] + recv_a1[s, :]
            p2a[c].start()
            p1b[c].wait()
            acc_b[s, :] = x_ref[pl.ds(b_mine + c * h, h), :] + recv_b1[s, :]
            p2b[c].start()

        p3 = []
        for c in range(2):
            s = pl.ds(c * h, h)
            p2a[c].wait()
            out_ref[pl.ds(a_mine + c * h, h), :] = acc_a[s, :] + recv_a2[s, :]
            r = exch(out_ref.at[pl.ds(a_mine + c * h, h), :],
                     out_ref.at[pl.ds(a_mine + c * h, h), :], 8 + c, x_nbr)
            r.start()
            p3.append(r)
            p2b[c].wait()
            out_ref[pl.ds(b_mine + c * h, h), :] = acc_b[s, :] + recv_b2[s, :]
            r = exch(out_ref.at[pl.ds(b_mine + c * h, h), :],
                     out_ref.at[pl.ds(b_mine + c * h, h), :], 10 + c, y_nbr)
            r.start()
            p3.append(r)
        for r in p3:
            r.wait()

    return pl.pallas_call(
        body,
        out_shape=jax.ShapeDtypeStruct((m, n), jnp.bfloat16),
        in_specs=[pl.BlockSpec(memory_space=pltpu.VMEM)],
        out_specs=pl.BlockSpec(memory_space=pltpu.VMEM),
        input_output_aliases={0: 0},
        scratch_shapes=[
            pltpu.VMEM((q, n), jnp.bfloat16),
            pltpu.VMEM((q, n), jnp.bfloat16),
            pltpu.VMEM((q, n), jnp.bfloat16),
            pltpu.VMEM((q, n), jnp.bfloat16),
            pltpu.VMEM((q, n), jnp.bfloat16),
            pltpu.VMEM((q, n), jnp.bfloat16),
            pltpu.SemaphoreType.DMA((12,)),
            pltpu.SemaphoreType.DMA((12,)),
        ],
        compiler_params=pltpu.CompilerParams(
            collective_id=0,
            vmem_limit_bytes=100 * 1024 * 1024,
        ),
    )(xs)


# device time: 161705 ns/iter; 1.1289x vs baseline; 1.1289x over previous
import jax
import jax.numpy as jnp
from jax import lax
from jax.experimental import pallas as pl
from jax.experimental.pallas import tpu as pltpu


def kernel(x):
    xs32 = x[0, 0]
    m, n = xs32.shape
    q = m // 4
    h = q // 2

    def body(
        x_hbm,
        out_ref,
        recv_a1,
        recv_b1,
        recv_a2,
        recv_b2,
        send_a,
        send_b,
        mine_a,
        mine_b,
        pool,
        send_sems,
        recv_sems,
        load_sems,
    ):
        mx = lax.axis_index("x")
        my = lax.axis_index("y")
        x_nbr = (1 - mx, my)
        y_nbr = (mx, 1 - my)

        a_mine = mx * q
        a_theirs = (1 - mx) * q
        b_mine = 2 * q + my * q
        b_theirs = 2 * q + (1 - my) * q

        barrier = pltpu.get_barrier_semaphore()
        for nbr in (x_nbr, y_nbr):
            pl.semaphore_signal(
                barrier, inc=1, device_id=nbr,
                device_id_type=pl.DeviceIdType.MESH,
            )
        pl.semaphore_wait(barrier, 2)

        def exch(src, dst, sem, nbr):
            return pltpu.make_async_remote_copy(
                src_ref=src, dst_ref=dst,
                send_sem=send_sems.at[sem], recv_sem=recv_sems.at[sem],
                device_id=nbr, device_id_type=pl.DeviceIdType.MESH,
            )

        loads = [
            (a_theirs, send_a), (b_theirs, send_b),
            (a_theirs + h, send_a), (b_theirs + h, send_b),
            (a_mine, mine_a), (b_mine, mine_b),
            (a_mine + h, mine_a), (b_mine + h, mine_b),
        ]

        def start_load(k):
            row, _ = loads[k]
            cp = pltpu.make_async_copy(
                x_hbm.at[pl.ds(row, h), :],
                pool.at[k % 2],
                load_sems.at[k % 2],
            )
            cp.start()
            return cp

        p1a = [
            exch(send_a.at[pl.ds(c * h, h), :],
                 recv_a1.at[pl.ds(c * h, h), :], 0 + c, x_nbr)
            for c in range(2)
        ]
        p1b = [
            exch(send_b.at[pl.ds(c * h, h), :],
                 recv_b1.at[pl.ds(c * h, h), :], 2 + c, y_nbr)
            for c in range(2)
        ]

        pend = [start_load(0), start_load(1)]
        for k in range(8):
            pend[k % 2].wait()
            dst_off = (k // 2 % 2) * h
            _, dst = loads[k]
            dst[pl.ds(dst_off, h), :] = pool[k % 2].astype(jnp.bfloat16)
            if k + 2 < 8:
                pend[k % 2] = start_load(k + 2)
            if k == 0:
                p1a[0].start()
            elif k == 1:
                p1b[0].start()
            elif k == 2:
                p1a[1].start()
            elif k == 3:
                p1b[1].start()

        p2a = [
            exch(out_ref.at[pl.ds(a_mine + c * h, h), :],
                 recv_a2.at[pl.ds(c * h, h), :], 4 + c, y_nbr)
            for c in range(2)
        ]
        p2b = [
            exch(out_ref.at[pl.ds(b_mine + c * h, h), :],
                 recv_b2.at[pl.ds(c * h, h), :], 6 + c, x_nbr)
            for c in range(2)
        ]
        for c in range(2):
            s = pl.ds(c * h, h)
            p1a[c].wait()
            out_ref[pl.ds(a_mine + c * h, h), :] = mine_a[s, :] + recv_a1[s, :]
            p2a[c].start()
            p1b[c].wait()
            out_ref[pl.ds(b_mine + c * h, h), :] = mine_b[s, :] + recv_b1[s, :]
            p2b[c].start()

        p3 = []
        for c in range(2):
            s = pl.ds(c * h, h)
            ra = pl.ds(a_mine + c * h, h)
            rb = pl.ds(b_mine + c * h, h)
            p2a[c].wait()
            out_ref[ra, :] = out_ref[ra, :] + recv_a2[s, :]
            r = exch(out_ref.at[ra, :], out_ref.at[ra, :], 8 + c, x_nbr)
            r.start()
            p3.append(r)
            p2b[c].wait()
            out_ref[rb, :] = out_ref[rb, :] + recv_b2[s, :]
            r = exch(out_ref.at[rb, :], out_ref.at[rb, :], 10 + c, y_nbr)
            r.start()
            p3.append(r)
        for r in p3:
            r.wait()

    return pl.pallas_call(
        body,
        out_shape=jax.ShapeDtypeStruct((m, n), jnp.bfloat16),
        in_specs=[pl.BlockSpec(memory_space=pl.ANY)],
        out_specs=pl.BlockSpec(memory_space=pltpu.VMEM),
        scratch_shapes=[
            pltpu.VMEM((q, n), jnp.bfloat16),
            pltpu.VMEM((q, n), jnp.bfloat16),
            pltpu.VMEM((q, n), jnp.bfloat16),
            pltpu.VMEM((q, n), jnp.bfloat16),
            pltpu.VMEM((q, n), jnp.bfloat16),
            pltpu.VMEM((q, n), jnp.bfloat16),
            pltpu.VMEM((q, n), jnp.bfloat16),
            pltpu.VMEM((q, n), jnp.bfloat16),
            pltpu.VMEM((2, h, n), jnp.float32),
            pltpu.SemaphoreType.DMA((12,)),
            pltpu.SemaphoreType.DMA((12,)),
            pltpu.SemaphoreType.DMA((2,)),
        ],
        compiler_params=pltpu.CompilerParams(
            collective_id=0,
            vmem_limit_bytes=100 * 1024 * 1024,
        ),
    )(xs32)


# device time: 157365 ns/iter; 1.1600x vs baseline; 1.0276x over previous
import jax
import jax.numpy as jnp
from jax import lax
from jax.experimental import pallas as pl
from jax.experimental.pallas import tpu as pltpu


def kernel(x):
    xs32 = x[0, 0]
    m, n = xs32.shape
    q = m // 4
    h = q // 2

    def body(
        x_hbm,
        out_ref,
        recv_a1,
        recv_b1,
        recv_a2,
        recv_b2,
        send_a,
        send_b,
        mine_a,
        mine_b,
        acc_a,
        acc_b,
        fin_a,
        fin_b,
        pool,
        send_sems,
        recv_sems,
        load_sems,
        store_sems,
    ):
        mx = lax.axis_index("x")
        my = lax.axis_index("y")
        x_nbr = (1 - mx, my)
        y_nbr = (mx, 1 - my)

        a_mine = mx * q
        a_theirs = (1 - mx) * q
        b_mine = 2 * q + my * q
        b_theirs = 2 * q + (1 - my) * q

        barrier = pltpu.get_barrier_semaphore()
        for nbr in (x_nbr, y_nbr):
            pl.semaphore_signal(
                barrier, inc=1, device_id=nbr,
                device_id_type=pl.DeviceIdType.MESH,
            )
        pl.semaphore_wait(barrier, 2)

        def exch(src, dst, sem, nbr):
            return pltpu.make_async_remote_copy(
                src_ref=src, dst_ref=dst,
                send_sem=send_sems.at[sem], recv_sem=recv_sems.at[sem],
                device_id=nbr, device_id_type=pl.DeviceIdType.MESH,
            )

        loads = [
            (a_theirs, send_a), (b_theirs, send_b),
            (a_theirs + h, send_a), (b_theirs + h, send_b),
            (a_mine, mine_a), (b_mine, mine_b),
            (a_mine + h, mine_a), (b_mine + h, mine_b),
        ]

        def start_load(k):
            row, _ = loads[k]
            cp = pltpu.make_async_copy(
                x_hbm.at[pl.ds(row, h), :],
                pool.at[k % 2],
                load_sems.at[k % 2],
            )
            cp.start()
            return cp

        p1a = [
            exch(send_a.at[pl.ds(c * h, h), :],
                 recv_a1.at[pl.ds(c * h, h), :], 0 + c, x_nbr)
            for c in range(2)
        ]
        p1b = [
            exch(send_b.at[pl.ds(c * h, h), :],
                 recv_b1.at[pl.ds(c * h, h), :], 2 + c, y_nbr)
            for c in range(2)
        ]

        pend = [start_load(0), start_load(1)]
        for k in range(8):
            pend[k % 2].wait()
            dst_off = (k // 2 % 2) * h
            _, dst = loads[k]
            dst[pl.ds(dst_off, h), :] = pool[k % 2].astype(jnp.bfloat16)
            if k + 2 < 8:
                pend[k % 2] = start_load(k + 2)
            if k == 0:
                p1a[0].start()
            elif k == 1:
                p1b[0].start()
            elif k == 2:
                p1a[1].start()
            elif k == 3:
                p1b[1].start()

        p2a = [
            exch(acc_a.at[pl.ds(c * h, h), :],
                 recv_a2.at[pl.ds(c * h, h), :], 4 + c, y_nbr)
            for c in range(2)
        ]
        p2b = [
            exch(acc_b.at[pl.ds(c * h, h), :],
                 recv_b2.at[pl.ds(c * h, h), :], 6 + c, x_nbr)
            for c in range(2)
        ]
        for c in range(2):
            s = pl.ds(c * h, h)
            p1a[c].wait()
            acc_a[s, :] = mine_a[s, :] + recv_a1[s, :]
            p2a[c].start()
            p1b[c].wait()
            acc_b[s, :] = mine_b[s, :] + recv_b1[s, :]
            p2b[c].start()

        p3 = []
        stores = []
        for c in range(2):
            s = pl.ds(c * h, h)
            ra = pl.ds(a_mine + c * h, h)
            rb = pl.ds(b_mine + c * h, h)
            p2a[c].wait()
            fin_a[s, :] = acc_a[s, :] + recv_a2[s, :]
            r = exch(fin_a.at[s, :], out_ref.at[ra, :], 8 + c, x_nbr)
            r.start()
            p3.append(r)
            cp = pltpu.make_async_copy(
                fin_a.at[s, :], out_ref.at[ra, :], store_sems.at[2 * c]
            )
            cp.start()
            stores.append(cp)
            p2b[c].wait()
            fin_b[s, :] = acc_b[s, :] + recv_b2[s, :]
            r = exch(fin_b.at[s, :], out_ref.at[rb, :], 10 + c, y_nbr)
            r.start()
            p3.append(r)
            cp = pltpu.make_async_copy(
                fin_b.at[s, :], out_ref.at[rb, :], store_sems.at[2 * c + 1]
            )
            cp.start()
            stores.append(cp)
        for cp in stores:
            cp.wait()
        for r in p3:
            r.wait()

    return pl.pallas_call(
        body,
        out_shape=jax.ShapeDtypeStruct((m, n), jnp.bfloat16),
        in_specs=[pl.BlockSpec(memory_space=pl.ANY)],
        out_specs=pl.BlockSpec(memory_space=pl.ANY),
        scratch_shapes=[
            pltpu.VMEM((q, n), jnp.bfloat16),
            pltpu.VMEM((q, n), jnp.bfloat16),
            pltpu.VMEM((q, n), jnp.bfloat16),
            pltpu.VMEM((q, n), jnp.bfloat16),
            pltpu.VMEM((q, n), jnp.bfloat16),
            pltpu.VMEM((q, n), jnp.bfloat16),
            pltpu.VMEM((q, n), jnp.bfloat16),
            pltpu.VMEM((q, n), jnp.bfloat16),
            pltpu.VMEM((q, n), jnp.bfloat16),
            pltpu.VMEM((q, n), jnp.bfloat16),
            pltpu.VMEM((q, n), jnp.bfloat16),
            pltpu.VMEM((q, n), jnp.bfloat16),
            pltpu.VMEM((2, h, n), jnp.float32),
            pltpu.SemaphoreType.DMA((12,)),
            pltpu.SemaphoreType.DMA((12,)),
            pltpu.SemaphoreType.DMA((2,)),
            pltpu.SemaphoreType.DMA((4,)),
        ],
        compiler_params=pltpu.CompilerParams(
            collective_id=0,
            vmem_limit_bytes=100 * 1024 * 1024,
        ),
    )(xs32)


# device time: 156160 ns/iter; 1.1690x vs baseline; 1.0077x over previous
import jax
import jax.numpy as jnp
from jax import lax
from jax.experimental import pallas as pl
from jax.experimental.pallas import tpu as pltpu

C = 4


def kernel(x):
    xs32 = x[0, 0]
    m, n = xs32.shape
    q = m // 4
    h = q // C

    def body(
        x_hbm,
        out_ref,
        recv_a1,
        recv_b1,
        recv_a2,
        recv_b2,
        send_a,
        send_b,
        mine_a,
        mine_b,
        acc_a,
        acc_b,
        fin_a,
        fin_b,
        pool,
        send_sems,
        recv_sems,
        load_sems,
        store_sems,
    ):
        mx = lax.axis_index("x")
        my = lax.axis_index("y")
        x_nbr = (1 - mx, my)
        y_nbr = (mx, 1 - my)

        a_mine = mx * q
        a_theirs = (1 - mx) * q
        b_mine = 2 * q + my * q
        b_theirs = 2 * q + (1 - my) * q

        barrier = pltpu.get_barrier_semaphore()
        for nbr in (x_nbr, y_nbr):
            pl.semaphore_signal(
                barrier, inc=1, device_id=nbr,
                device_id_type=pl.DeviceIdType.MESH,
            )
        pl.semaphore_wait(barrier, 2)

        def sem_idx(phase, stream, c):
            return phase * 2 * C + stream * C + c

        def exch(src, dst, phase, stream, c, nbr):
            i = sem_idx(phase, stream, c)
            return pltpu.make_async_remote_copy(
                src_ref=src, dst_ref=dst,
                send_sem=send_sems.at[i], recv_sem=recv_sems.at[i],
                device_id=nbr, device_id_type=pl.DeviceIdType.MESH,
            )

        loads = []
        for c in range(C):
            loads.append((a_theirs + c * h, send_a, c))
            loads.append((b_theirs + c * h, send_b, c))
        for c in range(C):
            loads.append((a_mine + c * h, mine_a, c))
            loads.append((b_mine + c * h, mine_b, c))

        def start_load(k):
            row, _, _ = loads[k]
            cp = pltpu.make_async_copy(
                x_hbm.at[pl.ds(row, h), :],
                pool.at[k % 2],
                load_sems.at[k % 2],
            )
            cp.start()
            return cp

        p1a = [
            exch(send_a.at[pl.ds(c * h, h), :],
                 recv_a1.at[pl.ds(c * h, h), :], 0, 0, c, x_nbr)
            for c in range(C)
        ]
        p1b = [
            exch(send_b.at[pl.ds(c * h, h), :],
                 recv_b1.at[pl.ds(c * h, h), :], 0, 1, c, y_nbr)
            for c in range(C)
        ]

        pend = [start_load(0), start_load(1)]
        for k in range(2 * 2 * C):
            pend[k % 2].wait()
            _, dst, c = loads[k]
            dst[pl.ds(c * h, h), :] = pool[k % 2].astype(jnp.bfloat16)
            if k + 2 < 2 * 2 * C:
                pend[k % 2] = start_load(k + 2)
            if k < 2 * C:
                if k % 2 == 0:
                    p1a[k // 2].start()
                else:
                    p1b[k // 2].start()

        p2a = [
            exch(acc_a.at[pl.ds(c * h, h), :],
                 recv_a2.at[pl.ds(c * h, h), :], 1, 0, c, y_nbr)
            for c in range(C)
        ]
        p2b = [
            exch(acc_b.at[pl.ds(c * h, h), :],
                 recv_b2.at[pl.ds(c * h, h), :], 1, 1, c, x_nbr)
            for c in range(C)
        ]
        for c in range(C):
            s = pl.ds(c * h, h)
            p1a[c].wait()
            acc_a[s, :] = mine_a[s, :] + recv_a1[s, :]
            p2a[c].start()
            p1b[c].wait()
            acc_b[s, :] = mine_b[s, :] + recv_b1[s, :]
            p2b[c].start()

        p3 = []
        stores = []
        for c in range(C):
            s = pl.ds(c * h, h)
            ra = pl.ds(a_mine + c * h, h)
            rb = pl.ds(b_mine + c * h, h)
            p2a[c].wait()
            fin_a[s, :] = acc_a[s, :] + recv_a2[s, :]
            r = exch(fin_a.at[s, :], out_ref.at[ra, :], 2, 0, c, x_nbr)
            r.start()
            p3.append(r)
            cp = pltpu.make_async_copy(
                fin_a.at[s, :], out_ref.at[ra, :], store_sems.at[2 * c]
            )
            cp.start()
            stores.append(cp)
            p2b[c].wait()
            fin_b[s, :] = acc_b[s, :] + recv_b2[s, :]
            r = exch(fin_b.at[s, :], out_ref.at[rb, :], 2, 1, c, y_nbr)
            r.start()
            p3.append(r)
            cp = pltpu.make_async_copy(
                fin_b.at[s, :], out_ref.at[rb, :], store_sems.at[2 * c + 1]
            )
            cp.start()
            stores.append(cp)
        for cp in stores:
            cp.wait()
        for r in p3:
            r.wait()

    return pl.pallas_call(
        body,
        out_shape=jax.ShapeDtypeStruct((m, n), jnp.bfloat16),
        in_specs=[pl.BlockSpec(memory_space=pl.ANY)],
        out_specs=pl.BlockSpec(memory_space=pl.ANY),
        scratch_shapes=[
            pltpu.VMEM((q, n), jnp.bfloat16),
            pltpu.VMEM((q, n), jnp.bfloat16),
            pltpu.VMEM((q, n), jnp.bfloat16),
            pltpu.VMEM((q, n), jnp.bfloat16),
            pltpu.VMEM((q, n), jnp.bfloat16),
            pltpu.VMEM((q, n), jnp.bfloat16),
            pltpu.VMEM((q, n), jnp.bfloat16),
            pltpu.VMEM((q, n), jnp.bfloat16),
            pltpu.VMEM((q, n), jnp.bfloat16),
            pltpu.VMEM((q, n), jnp.bfloat16),
            pltpu.VMEM((q, n), jnp.bfloat16),
            pltpu.VMEM((q, n), jnp.bfloat16),
            pltpu.VMEM((2, h, n), jnp.float32),
            pltpu.SemaphoreType.DMA((3 * 2 * C,)),
            pltpu.SemaphoreType.DMA((3 * 2 * C,)),
            pltpu.SemaphoreType.DMA((2,)),
            pltpu.SemaphoreType.DMA((2 * C,)),
        ],
        compiler_params=pltpu.CompilerParams(
            collective_id=0,
            vmem_limit_bytes=100 * 1024 * 1024,
        ),
    )(xs32)


# device time: 155961 ns/iter; 1.1705x vs baseline; 1.0013x over previous
import jax
import jax.numpy as jnp
from jax import lax
from jax.experimental import pallas as pl
from jax.experimental.pallas import tpu as pltpu

C = 8


def kernel(x):
    xs32 = x[0, 0]
    m, n = xs32.shape
    q = m // 4
    h = q // C

    def body(
        x_hbm,
        out_ref,
        recv_a1,
        recv_b1,
        recv_a2,
        recv_b2,
        send_a,
        send_b,
        mine_a,
        mine_b,
        acc_a,
        acc_b,
        fin_a,
        fin_b,
        pool,
        send_sems,
        recv_sems,
        load_sems,
        store_sems,
    ):
        mx = lax.axis_index("x")
        my = lax.axis_index("y")
        x_nbr = (1 - mx, my)
        y_nbr = (mx, 1 - my)

        a_mine = mx * q
        a_theirs = (1 - mx) * q
        b_mine = 2 * q + my * q
        b_theirs = 2 * q + (1 - my) * q

        barrier = pltpu.get_barrier_semaphore()
        for nbr in (x_nbr, y_nbr):
            pl.semaphore_signal(
                barrier, inc=1, device_id=nbr,
                device_id_type=pl.DeviceIdType.MESH,
            )
        pl.semaphore_wait(barrier, 2)

        def sem_idx(phase, stream, c):
            return phase * 2 * C + stream * C + c

        def exch(src, dst, phase, stream, c, nbr):
            i = sem_idx(phase, stream, c)
            return pltpu.make_async_remote_copy(
                src_ref=src, dst_ref=dst,
                send_sem=send_sems.at[i], recv_sem=recv_sems.at[i],
                device_id=nbr, device_id_type=pl.DeviceIdType.MESH,
            )

        loads = []
        for c in range(C):
            loads.append((a_theirs + c * h, send_a, c))
            loads.append((b_theirs + c * h, send_b, c))
        for c in range(C):
            loads.append((a_mine + c * h, mine_a, c))
            loads.append((b_mine + c * h, mine_b, c))

        def start_load(k):
            row, _, _ = loads[k]
            cp = pltpu.make_async_copy(
                x_hbm.at[pl.ds(row, h), :],
                pool.at[k % 2],
                load_sems.at[k % 2],
            )
            cp.start()
            return cp

        p1a = [
            exch(send_a.at[pl.ds(c * h, h), :],
                 recv_a1.at[pl.ds(c * h, h), :], 0, 0, c, x_nbr)
            for c in range(C)
        ]
        p1b = [
            exch(send_b.at[pl.ds(c * h, h), :],
                 recv_b1.at[pl.ds(c * h, h), :], 0, 1, c, y_nbr)
            for c in range(C)
        ]

        pend = [start_load(0), start_load(1)]
        for k in range(2 * 2 * C):
            pend[k % 2].wait()
            _, dst, c = loads[k]
            dst[pl.ds(c * h, h), :] = pool[k % 2].astype(jnp.bfloat16)
            if k + 2 < 2 * 2 * C:
                pend[k % 2] = start_load(k + 2)
            if k < 2 * C:
                if k % 2 == 0:
                    p1a[k // 2].start()
                else:
                    p1b[k // 2].start()

        p2a = [
            exch(acc_a.at[pl.ds(c * h, h), :],
                 recv_a2.at[pl.ds(c * h, h), :], 1, 0, c, y_nbr)
            for c in range(C)
        ]
        p2b = [
            exch(acc_b.at[pl.ds(c * h, h), :],
                 recv_b2.at[pl.ds(c * h, h), :], 1, 1, c, x_nbr)
            for c in range(C)
        ]
        for c in range(C):
            s = pl.ds(c * h, h)
            p1a[c].wait()
            acc_a[s, :] = mine_a[s, :] + recv_a1[s, :]
            p2a[c].start()
            p1b[c].wait()
            acc_b[s, :] = mine_b[s, :] + recv_b1[s, :]
            p2b[c].start()

        p3 = []
        stores = []
        for c in range(C):
            s = pl.ds(c * h, h)
            ra = pl.ds(a_mine + c * h, h)
            rb = pl.ds(b_mine + c * h, h)
            p2a[c].wait()
            fin_a[s, :] = acc_a[s, :] + recv_a2[s, :]
            r = exch(fin_a.at[s, :], out_ref.at[ra, :], 2, 0, c, x_nbr)
            r.start()
            p3.append(r)
            cp = pltpu.make_async_copy(
                fin_a.at[s, :], out_ref.at[ra, :], store_sems.at[2 * c]
            )
            cp.start()
            stores.append(cp)
            p2b[c].wait()
            fin_b[s, :] = acc_b[s, :] + recv_b2[s, :]
            r = exch(fin_b.at[s, :], out_ref.at[rb, :], 2, 1, c, y_nbr)
            r.start()
            p3.append(r)
            cp = pltpu.make_async_copy(
                fin_b.at[s, :], out_ref.at[rb, :], store_sems.at[2 * c + 1]
            )
            cp.start()
            stores.append(cp)
        for cp in stores:
            cp.wait()
        for r in p3:
            r.wait()

    return pl.pallas_call(
        body,
        out_shape=jax.ShapeDtypeStruct((m, n), jnp.bfloat16),
        in_specs=[pl.BlockSpec(memory_space=pl.ANY)],
        out_specs=pl.BlockSpec(memory_space=pl.ANY),
        scratch_shapes=[
            pltpu.VMEM((q, n), jnp.bfloat16),
            pltpu.VMEM((q, n), jnp.bfloat16),
            pltpu.VMEM((q, n), jnp.bfloat16),
            pltpu.VMEM((q, n), jnp.bfloat16),
            pltpu.VMEM((q, n), jnp.bfloat16),
            pltpu.VMEM((q, n), jnp.bfloat16),
            pltpu.VMEM((q, n), jnp.bfloat16),
            pltpu.VMEM((q, n), jnp.bfloat16),
            pltpu.VMEM((q, n), jnp.bfloat16),
            pltpu.VMEM((q, n), jnp.bfloat16),
            pltpu.VMEM((q, n), jnp.bfloat16),
            pltpu.VMEM((q, n), jnp.bfloat16),
            pltpu.VMEM((2, h, n), jnp.float32),
            pltpu.SemaphoreType.DMA((3 * 2 * C,)),
            pltpu.SemaphoreType.DMA((3 * 2 * C,)),
            pltpu.SemaphoreType.DMA((2,)),
            pltpu.SemaphoreType.DMA((2 * C,)),
        ],
        compiler_params=pltpu.CompilerParams(
            collective_id=0,
            vmem_limit_bytes=100 * 1024 * 1024,
        ),
    )(xs32)
